# Initial kernel scaffold; baseline (speedup 1.0000x reference)
import jax
import jax.numpy as jnp
from jax.experimental import pallas as pl


def kernel(x, W, b):
    raise NotImplementedError("write your pallas kernel here")


# speedup vs baseline: 5.0707x; 5.0707x over previous
import jax
import jax.numpy as jnp
from jax.experimental import pallas as pl

_TOKENS = 16384
_D = 4096
_E = 64
_K = 8
_BT = 512


def _gate_kernel(x_ref, wt_ref, b_ref, out_ref):
    logits = (
        jnp.dot(x_ref[:], wt_ref[:], preferred_element_type=jnp.float32)
        + b_ref[:]
    )
    col = jax.lax.broadcasted_iota(jnp.int32, logits.shape, 1)
    cur = logits
    mask = jnp.zeros_like(logits)
    for _ in range(_K):
        m = jnp.max(cur, axis=1, keepdims=True)
        is_max = cur == m
        pick = jnp.min(jnp.where(is_max, col, _E), axis=1, keepdims=True)
        sel = col == pick
        mask = jnp.where(sel, 1.0, mask)
        cur = jnp.where(sel, -jnp.inf, cur)
    row_max = jnp.max(logits, axis=1, keepdims=True)
    e = jnp.exp(logits - row_max)
    z = jnp.sum(e, axis=1, keepdims=True)
    s = jnp.sum(e * mask, axis=1, keepdims=True)
    out_ref[:] = (e * mask) / (s + 1e-6 * z)


def kernel(x, W, b):
    wt = W.T
    b2 = b.reshape(1, _E)
    return pl.pallas_call(
        _gate_kernel,
        grid=(_TOKENS // _BT,),
        in_specs=[
            pl.BlockSpec((_BT, _D), lambda i: (i, 0)),
            pl.BlockSpec((_D, _E), lambda i: (0, 0)),
            pl.BlockSpec((1, _E), lambda i: (0, 0)),
        ],
        out_specs=pl.BlockSpec((_BT, _E), lambda i: (i, 0)),
        out_shape=jax.ShapeDtypeStruct((_TOKENS, _E), jnp.float32),
    )(x, wt, b2)


# speedup vs baseline: 6.2293x; 1.2285x over previous
import jax
import jax.numpy as jnp
from jax.experimental import pallas as pl
from jax.experimental.pallas import tpu as pltpu

_TOKENS = 16384
_D = 4096
_E = 64
_K = 8
_BT = 512


def _gate_kernel(x_ref, wt_ref, b_ref, out_ref):
    logits = (
        jnp.dot(x_ref[:], wt_ref[:], preferred_element_type=jnp.float32)
        + b_ref[:]
    )
    cur = logits
    mask = jnp.zeros_like(logits)
    for _ in range(_K):
        m = jnp.max(cur, axis=1, keepdims=True)
        sel = cur >= m
        mask = jnp.where(sel, 1.0, mask)
        cur = jnp.where(sel, -jnp.inf, cur)
    row_max = jnp.max(logits, axis=1, keepdims=True)
    e = jnp.exp(logits - row_max)
    z = jnp.sum(e, axis=1, keepdims=True)
    s = jnp.sum(e * mask, axis=1, keepdims=True)
    out_ref[:] = (e * mask) / (s + 1e-6 * z)


def kernel(x, W, b):
    wt = W.T
    b2 = b.reshape(1, _E)
    return pl.pallas_call(
        _gate_kernel,
        grid=(_TOKENS // _BT,),
        in_specs=[
            pl.BlockSpec((_BT, _D), lambda i: (i, 0)),
            pl.BlockSpec((_D, _E), lambda i: (0, 0)),
            pl.BlockSpec((1, _E), lambda i: (0, 0)),
        ],
        out_specs=pl.BlockSpec((_BT, _E), lambda i: (i, 0)),
        out_shape=jax.ShapeDtypeStruct((_TOKENS, _E), jnp.float32),
        compiler_params=pltpu.CompilerParams(
            dimension_semantics=("parallel",),
        ),
    )(x, wt, b2)


# speedup vs baseline: 6.7389x; 1.0818x over previous
import jax
import jax.numpy as jnp
from jax.experimental import pallas as pl
from jax.experimental.pallas import tpu as pltpu

_TOKENS = 16384
_D = 4096
_E = 64
_K = 8
_BT = 1024


def _gate_kernel(x_ref, wt_ref, b_ref, out_ref):
    logits = (
        jnp.dot(x_ref[:], wt_ref[:], preferred_element_type=jnp.float32)
        + b_ref[:]
    )
    cur = logits
    mask = jnp.zeros_like(logits)
    for _ in range(_K):
        m = jnp.max(cur, axis=1, keepdims=True)
        sel = cur >= m
        mask = jnp.where(sel, 1.0, mask)
        cur = jnp.where(sel, -jnp.inf, cur)
    row_max = jnp.max(logits, axis=1, keepdims=True)
    e = jnp.exp(logits - row_max)
    z = jnp.sum(e, axis=1, keepdims=True)
    s = jnp.sum(e * mask, axis=1, keepdims=True)
    out_ref[:] = (e * mask) / (s + 1e-6 * z)


def kernel(x, W, b):
    wt = W.T
    b2 = b.reshape(1, _E)
    return pl.pallas_call(
        _gate_kernel,
        grid=(_TOKENS // _BT,),
        in_specs=[
            pl.BlockSpec((_BT, _D), lambda i: (i, 0)),
            pl.BlockSpec((_D, _E), lambda i: (0, 0)),
            pl.BlockSpec((1, _E), lambda i: (0, 0)),
        ],
        out_specs=pl.BlockSpec((_BT, _E), lambda i: (i, 0)),
        out_shape=jax.ShapeDtypeStruct((_TOKENS, _E), jnp.float32),
        compiler_params=pltpu.CompilerParams(
            dimension_semantics=("parallel",),
        ),
    )(x, wt, b2)


# speedup vs baseline: 6.7390x; 1.0000x over previous
import jax
import jax.numpy as jnp
from jax.experimental import pallas as pl
from jax.experimental.pallas import tpu as pltpu

_TOKENS = 16384
_D = 4096
_E = 64
_K = 8
_BT = 1024
_DH = _D // 2


def _gate_kernel(x0_ref, x1_ref, wt0_ref, wt1_ref, b_ref, out_ref):
    logits = (
        jnp.dot(x0_ref[:], wt0_ref[:], preferred_element_type=jnp.float32)
        + jnp.dot(x1_ref[:], wt1_ref[:], preferred_element_type=jnp.float32)
        + b_ref[:]
    )
    cur = logits
    mask = jnp.zeros_like(logits)
    for _ in range(_K):
        m = jnp.max(cur, axis=1, keepdims=True)
        sel = cur >= m
        mask = jnp.where(sel, 1.0, mask)
        cur = jnp.where(sel, -jnp.inf, cur)
    row_max = jnp.max(logits, axis=1, keepdims=True)
    e = jnp.exp(logits - row_max)
    z = jnp.sum(e, axis=1, keepdims=True)
    s = jnp.sum(e * mask, axis=1, keepdims=True)
    out_ref[:] = (e * mask) / (s + 1e-6 * z)


def kernel(x, W, b):
    wt = W.T
    b2 = b.reshape(1, _E)
    return pl.pallas_call(
        _gate_kernel,
        grid=(_TOKENS // _BT,),
        in_specs=[
            pl.BlockSpec((_BT, _DH), lambda i: (i, 0)),
            pl.BlockSpec((_BT, _DH), lambda i: (i, 1)),
            pl.BlockSpec((_DH, _E), lambda i: (0, 0)),
            pl.BlockSpec((_DH, _E), lambda i: (1, 0)),
            pl.BlockSpec((1, _E), lambda i: (0, 0)),
        ],
        out_specs=pl.BlockSpec((_BT, _E), lambda i: (i, 0)),
        out_shape=jax.ShapeDtypeStruct((_TOKENS, _E), jnp.float32),
        compiler_params=pltpu.CompilerParams(
            dimension_semantics=("parallel",),
        ),
    )(x, x, wt, wt, b2)


# speedup vs baseline: 7.0195x; 1.0416x over previous
import jax
import jax.numpy as jnp
from jax.experimental import pallas as pl
from jax.experimental.pallas import tpu as pltpu

_TOKENS = 16384
_D = 4096
_E = 64
_K = 8
_BT = 1024


def _gate_kernel(x_ref, w_ref, b_ref, out_ref):
    logits = (
        jax.lax.dot_general(
            x_ref[:],
            w_ref[:],
            (((1,), (1,)), ((), ())),
            preferred_element_type=jnp.float32,
        )
        + b_ref[:]
    )
    cur = logits
    mask = jnp.zeros_like(logits)
    for _ in range(_K):
        m = jnp.max(cur, axis=1, keepdims=True)
        sel = cur >= m
        mask = jnp.where(sel, 1.0, mask)
        cur = jnp.where(sel, -jnp.inf, cur)
    row_max = jnp.max(logits, axis=1, keepdims=True)
    e = jnp.exp(logits - row_max)
    z = jnp.sum(e, axis=1, keepdims=True)
    s = jnp.sum(e * mask, axis=1, keepdims=True)
    out_ref[:] = (e * mask) / (s + 1e-6 * z)


def kernel(x, W, b):
    b2 = b.reshape(1, _E)
    return pl.pallas_call(
        _gate_kernel,
        grid=(_TOKENS // _BT,),
        in_specs=[
            pl.BlockSpec((_BT, _D), lambda i: (i, 0)),
            pl.BlockSpec((_E, _D), lambda i: (0, 0)),
            pl.BlockSpec((1, _E), lambda i: (0, 0)),
        ],
        out_specs=pl.BlockSpec((_BT, _E), lambda i: (i, 0)),
        out_shape=jax.ShapeDtypeStruct((_TOKENS, _E), jnp.float32),
        compiler_params=pltpu.CompilerParams(
            dimension_semantics=("parallel",),
        ),
    )(x, W, b2)
